# baseline (speedup 1.0000x reference)
import functools

import jax
import jax.numpy as jnp
from jax import lax
from jax.experimental import pallas as pl
from jax.experimental.pallas import tpu as pltpu
from jax.experimental.pallas import tpu_sc as plsc

_DIM = 300
_NW = 32
_CH = 128


def _body(idx_hbm, table_hbm, out_hbm, idx_v, buf0, buf1, sem0, sem1):
    nch = idx_hbm.shape[1]
    per_w = nch * _CH
    c = lax.axis_index("c")
    s = lax.axis_index("s")
    wid = s * 2 + c
    base = wid * per_w

    pltpu.sync_copy(idx_hbm.at[wid], idx_v)

    bufs = (buf0, buf1)
    sems = (sem0, sem1)

    pltpu.async_copy(table_hbm.at[idx_v.at[0]], buf0, sem0)
    pltpu.async_copy(table_hbm.at[idx_v.at[1]], buf1, sem1)

    @pl.loop(0, nch - 2, step=2)
    def _(jj):
        for b in range(2):
            j = jj + b
            pltpu.make_async_copy(
                table_hbm.at[idx_v.at[j]], bufs[b], sems[b]
            ).wait()
            pltpu.sync_copy(bufs[b], out_hbm.at[pl.ds(base + j * _CH, _CH)])
            pltpu.async_copy(table_hbm.at[idx_v.at[j + 2]], bufs[b], sems[b])

    for b in range(2):
        j = nch - 2 + b
        pltpu.make_async_copy(
            table_hbm.at[idx_v.at[j]], bufs[b], sems[b]
        ).wait()
        pltpu.sync_copy(bufs[b], out_hbm.at[pl.ds(base + j * _CH, _CH)])


@functools.partial(jax.jit, static_argnums=())
def _lookup(idx, table):
    total = idx.shape[0] * idx.shape[1] * idx.shape[2]
    nch = idx.shape[1]
    mesh = plsc.VectorSubcoreMesh(core_axis_name="c", subcore_axis_name="s")
    f = pl.kernel(
        _body,
        out_type=jax.ShapeDtypeStruct((total, _DIM), jnp.float32),
        mesh=mesh,
        scratch_types=[
            pltpu.VMEM((nch, _CH), jnp.int32),
            pltpu.VMEM((_CH, _DIM), jnp.float32),
            pltpu.VMEM((_CH, _DIM), jnp.float32),
            pltpu.SemaphoreType.DMA,
            pltpu.SemaphoreType.DMA,
        ],
        compiler_params=pltpu.CompilerParams(use_tc_tiling_on_sc=False),
    )
    return f(idx, table)


def kernel(inp, table):
    b, s = inp.shape
    total = b * s
    idx = inp.reshape(_NW, total // (_NW * _CH), _CH)
    out = _lookup(idx, table)
    return out.reshape(b, s, _DIM)


# speedup vs baseline: 1.3554x; 1.3554x over previous
import functools

import jax
import jax.numpy as jnp
from jax import lax
from jax.experimental import pallas as pl
from jax.experimental.pallas import tpu as pltpu
from jax.experimental.pallas import tpu_sc as plsc

_DIM = 300
_DIMP = 384
_NW = 32
_CH = 128


def _gather(table_hbm, idx_v, buf, sem, j):
    ii = idx_v.at[pl.ds(j * _CH, _CH)]
    for k in range(3):
        pltpu.async_copy(
            table_hbm.at[ii, pl.ds(128 * k, 128)],
            buf.at[:, pl.ds(128 * k, 128)],
            sem,
        )


def _wait_gather(table_hbm, idx_v, buf, sem, j):
    ii = idx_v.at[pl.ds(j * _CH, _CH)]
    for k in range(3):
        pltpu.make_async_copy(
            table_hbm.at[ii, pl.ds(128 * k, 128)],
            buf.at[:, pl.ds(128 * k, 128)],
            sem,
        ).wait()


def _body(idx_hbm, table_hbm, out_hbm, idx_v, buf0, buf1, sem0, sem1):
    per_w = idx_hbm.shape[0] // _NW
    nch = per_w // _CH
    c = lax.axis_index("c")
    s = lax.axis_index("s")
    wid = s * 2 + c
    base = wid * per_w

    pltpu.sync_copy(idx_hbm.at[pl.ds(base, per_w)], idx_v)

    bufs = (buf0, buf1)
    sems = (sem0, sem1)

    _gather(table_hbm, idx_v, buf0, sem0, 0)
    _gather(table_hbm, idx_v, buf1, sem1, 1)

    @pl.loop(0, nch - 2, step=2)
    def _(jj):
        for b in range(2):
            j = jj + b
            _wait_gather(table_hbm, idx_v, bufs[b], sems[b], j)
            pltpu.sync_copy(
                bufs[b], out_hbm.at[pl.ds(base + j * _CH, _CH), :]
            )
            _gather(table_hbm, idx_v, bufs[b], sems[b], j + 2)

    for b in range(2):
        j = nch - 2 + b
        _wait_gather(table_hbm, idx_v, bufs[b], sems[b], j)
        pltpu.sync_copy(bufs[b], out_hbm.at[pl.ds(base + j * _CH, _CH), :])


@jax.jit
def _lookup(idx, table_p):
    total = idx.shape[0]
    per_w = total // _NW
    mesh = plsc.VectorSubcoreMesh(core_axis_name="c", subcore_axis_name="s")
    f = pl.kernel(
        _body,
        out_type=jax.ShapeDtypeStruct((total, _DIMP), jnp.float32),
        mesh=mesh,
        scratch_types=[
            pltpu.VMEM((per_w,), jnp.int32),
            pltpu.VMEM((_CH, _DIMP), jnp.float32),
            pltpu.VMEM((_CH, _DIMP), jnp.float32),
            pltpu.SemaphoreType.DMA,
            pltpu.SemaphoreType.DMA,
        ],
    )
    return f(idx, table_p)


def kernel(inp, table):
    b, s = inp.shape
    idx = inp.reshape(b * s)
    table_p = jnp.pad(table, ((0, 0), (0, _DIMP - _DIM)))
    y = _lookup(idx, table_p)
    return y[:, :_DIM].reshape(b, s, _DIM)


# speedup vs baseline: 1.5843x; 1.1689x over previous
import functools

import jax
import jax.numpy as jnp
from jax import lax
from jax.experimental import pallas as pl
from jax.experimental.pallas import tpu as pltpu
from jax.experimental.pallas import tpu_sc as plsc

_DIM = 300
_DIMP = 384
_NW = 32
_CH = 128


def _gather(table_hbm, tail_hbm, idx_v, buf, sem, j):
    ii = idx_v.at[pl.ds(j * _CH, _CH)]
    pltpu.async_copy(
        table_hbm.at[ii, pl.ds(0, 256)], buf.at[:, pl.ds(0, 256)], sem
    )
    pltpu.async_copy(tail_hbm.at[ii], buf.at[:, pl.ds(256, 128)], sem)


def _wait_gather(table_hbm, tail_hbm, idx_v, buf, sem, j):
    ii = idx_v.at[pl.ds(j * _CH, _CH)]
    pltpu.make_async_copy(
        table_hbm.at[ii, pl.ds(0, 256)], buf.at[:, pl.ds(0, 256)], sem
    ).wait()
    pltpu.make_async_copy(
        tail_hbm.at[ii], buf.at[:, pl.ds(256, 128)], sem
    ).wait()


def _body(idx_hbm, table_hbm, tail_hbm, out_hbm, idx_v, buf0, buf1, sem0, sem1):
    per_w = idx_hbm.shape[0] // _NW
    nch = per_w // _CH
    c = lax.axis_index("c")
    s = lax.axis_index("s")
    wid = s * 2 + c
    base = wid * per_w

    pltpu.sync_copy(idx_hbm.at[pl.ds(base, per_w)], idx_v)

    bufs = (buf0, buf1)
    sems = (sem0, sem1)

    _gather(table_hbm, tail_hbm, idx_v, buf0, sem0, 0)
    _gather(table_hbm, tail_hbm, idx_v, buf1, sem1, 1)

    @pl.loop(0, nch - 2, step=2)
    def _(jj):
        for b in range(2):
            j = jj + b
            _wait_gather(table_hbm, tail_hbm, idx_v, bufs[b], sems[b], j)
            pltpu.sync_copy(
                bufs[b], out_hbm.at[pl.ds(base + j * _CH, _CH), :]
            )
            _gather(table_hbm, tail_hbm, idx_v, bufs[b], sems[b], j + 2)

    for b in range(2):
        j = nch - 2 + b
        _wait_gather(table_hbm, tail_hbm, idx_v, bufs[b], sems[b], j)
        pltpu.sync_copy(bufs[b], out_hbm.at[pl.ds(base + j * _CH, _CH), :])


@jax.jit
def _lookup(idx, table, tail):
    total = idx.shape[0]
    per_w = total // _NW
    mesh = plsc.VectorSubcoreMesh(core_axis_name="c", subcore_axis_name="s")
    f = pl.kernel(
        _body,
        out_type=jax.ShapeDtypeStruct((total, _DIMP), jnp.float32),
        mesh=mesh,
        scratch_types=[
            pltpu.VMEM((per_w,), jnp.int32),
            pltpu.VMEM((_CH, _DIMP), jnp.float32),
            pltpu.VMEM((_CH, _DIMP), jnp.float32),
            pltpu.SemaphoreType.DMA,
            pltpu.SemaphoreType.DMA,
        ],
    )
    return f(idx, table, tail)


def _fmt(y, b, s):
    def _fmt_body(x_ref, o_ref):
        o_ref[...] = x_ref[:, :_DIM].reshape(8, s, _DIM)

    return pl.pallas_call(
        _fmt_body,
        grid=(b // 8,),
        in_specs=[pl.BlockSpec((8 * s, _DIMP), lambda i: (i, 0))],
        out_specs=pl.BlockSpec((8, s, _DIM), lambda i: (i, 0, 0)),
        out_shape=jax.ShapeDtypeStruct((b, s, _DIM), jnp.float32),
    )(y)


def kernel(inp, table):
    b, s = inp.shape
    idx = inp.reshape(b * s)
    tail = jnp.pad(
        lax.slice(table, (0, 256), (table.shape[0], _DIM)),
        ((0, 0), (0, 128 - (_DIM - 256))),
    )
    y = _lookup(idx, table, tail)
    return _fmt(y, b, s)


# speedup vs baseline: 1.9393x; 1.2241x over previous
import functools

import jax
import jax.numpy as jnp
from jax import lax
from jax.experimental import pallas as pl
from jax.experimental.pallas import tpu as pltpu
from jax.experimental.pallas import tpu_sc as plsc

_DIM = 300
_DIMP = 384
_NW = 32
_CH = 128


def _gather(table_hbm, tail_hbm, idx_v, buf, sem, j):
    ii = idx_v.at[pl.ds(j * _CH, _CH)]
    pltpu.async_copy(
        table_hbm.at[ii, pl.ds(0, 256)], buf.at[:, pl.ds(0, 256)], sem
    )
    pltpu.async_copy(tail_hbm.at[ii], buf.at[:, pl.ds(256, 128)], sem)


def _wait_gather(table_hbm, tail_hbm, idx_v, buf, sem, j):
    ii = idx_v.at[pl.ds(j * _CH, _CH)]
    pltpu.make_async_copy(
        table_hbm.at[ii, pl.ds(0, 256)], buf.at[:, pl.ds(0, 256)], sem
    ).wait()
    pltpu.make_async_copy(
        tail_hbm.at[ii], buf.at[:, pl.ds(256, 128)], sem
    ).wait()


def _body(idx_hbm, table_hbm, tail_hbm, out_hbm, idx_v, buf0, buf1, sem0, sem1):
    per_w = idx_hbm.shape[0] // _NW
    nch = per_w // _CH
    c = lax.axis_index("c")
    s = lax.axis_index("s")
    wid = s * 2 + c
    base = wid * per_w

    pltpu.sync_copy(idx_hbm.at[pl.ds(base, per_w)], idx_v)

    bufs = (buf0, buf1)
    sems = (sem0, sem1)

    _gather(table_hbm, tail_hbm, idx_v, buf0, sem0, 0)
    _gather(table_hbm, tail_hbm, idx_v, buf1, sem1, 1)

    @pl.loop(0, nch - 2, step=2)
    def _(jj):
        for b in range(2):
            j = jj + b
            _wait_gather(table_hbm, tail_hbm, idx_v, bufs[b], sems[b], j)
            pltpu.sync_copy(
                bufs[b], out_hbm.at[pl.ds(base + j * _CH, _CH), :]
            )
            _gather(table_hbm, tail_hbm, idx_v, bufs[b], sems[b], j + 2)

    for b in range(2):
        j = nch - 2 + b
        _wait_gather(table_hbm, tail_hbm, idx_v, bufs[b], sems[b], j)
        pltpu.sync_copy(bufs[b], out_hbm.at[pl.ds(base + j * _CH, _CH), :])


@jax.jit
def _lookup(idx, table, tail):
    total = idx.shape[0]
    per_w = total // _NW
    mesh = plsc.VectorSubcoreMesh(core_axis_name="c", subcore_axis_name="s")
    f = pl.kernel(
        _body,
        out_type=jax.ShapeDtypeStruct((total, _DIMP), jnp.float32),
        mesh=mesh,
        scratch_types=[
            pltpu.VMEM((per_w,), jnp.int32),
            pltpu.VMEM((_CH, _DIMP), jnp.float32),
            pltpu.VMEM((_CH, _DIMP), jnp.float32),
            pltpu.SemaphoreType.DMA,
            pltpu.SemaphoreType.DMA,
        ],
    )
    return f(idx, table, tail)


def _fmt(y, b, s):
    def _fmt_body(x_ref, o_ref):
        o_ref[...] = x_ref[:, :_DIM].reshape(8, s, _DIM)

    return pl.pallas_call(
        _fmt_body,
        grid=(b // 8,),
        in_specs=[pl.BlockSpec((8 * s, _DIMP), lambda i: (i, 0))],
        out_specs=pl.BlockSpec((8, s, _DIM), lambda i: (i, 0, 0)),
        out_shape=jax.ShapeDtypeStruct((b, s, _DIM), jnp.float32),
    )(y)


def kernel(inp, table):
    b, s = inp.shape
    idx = inp.reshape(b * s)
    tail = jnp.pad(
        lax.slice(table, (0, 256), (table.shape[0], _DIM)),
        ((0, 0), (0, 128 - (_DIM - 256))),
    )
    y = _lookup(idx, table, tail)
    return y[:, :_DIM].reshape(b, s, _DIM)


# speedup vs baseline: 2.7199x; 1.4025x over previous
import functools

import jax
import jax.numpy as jnp
from jax import lax
from jax.experimental import pallas as pl
from jax.experimental.pallas import tpu as pltpu
from jax.experimental.pallas import tpu_sc as plsc

_DIM = 300
_DIMP = 384
_NW = 32
_BPW = 128


def _gather(table_hbm, tail_hbm, idx_row, asm, sem):
    pltpu.async_copy(
        table_hbm.at[idx_row, pl.ds(0, 256)], asm.at[:, pl.ds(0, 256)], sem
    )
    pltpu.async_copy(tail_hbm.at[idx_row], asm.at[:, pl.ds(256, 128)], sem)


def _wait_gather(table_hbm, tail_hbm, idx_row, asm, sem):
    pltpu.make_async_copy(
        table_hbm.at[idx_row, pl.ds(0, 256)], asm.at[:, pl.ds(0, 256)], sem
    ).wait()
    pltpu.make_async_copy(
        tail_hbm.at[idx_row], asm.at[:, pl.ds(256, 128)], sem
    ).wait()


def _body(inp_hbm, table_hbm, tail_hbm, out_hbm, idx_v, asm0, asm1, sem0, sem1):
    c = lax.axis_index("c")
    s = lax.axis_index("s")
    wid = s * 2 + c
    b0 = wid * _BPW

    pltpu.sync_copy(inp_hbm.at[pl.ds(b0, _BPW), :], idx_v)

    asms = (asm0, asm1)
    sems = (sem0, sem1)

    _gather(table_hbm, tail_hbm, idx_v.at[0], asm0, sem0)
    _gather(table_hbm, tail_hbm, idx_v.at[1], asm1, sem1)

    @pl.loop(0, _BPW - 2, step=2)
    def _(r):
        for p in range(2):
            _wait_gather(table_hbm, tail_hbm, idx_v.at[r + p], asms[p], sems[p])
            pltpu.sync_copy(asms[p], out_hbm.at[b0 + r + p])
            _gather(table_hbm, tail_hbm, idx_v.at[r + p + 2], asms[p], sems[p])

    for p in range(2):
        r = _BPW - 2 + p
        _wait_gather(table_hbm, tail_hbm, idx_v.at[r], asms[p], sems[p])
        pltpu.sync_copy(asms[p], out_hbm.at[b0 + r])


@jax.jit
def _lookup(inp, table, tail):
    b, s = inp.shape
    mesh = plsc.VectorSubcoreMesh(core_axis_name="c", subcore_axis_name="s")
    f = pl.kernel(
        _body,
        out_type=jax.ShapeDtypeStruct((b, s, _DIMP), jnp.float32),
        mesh=mesh,
        scratch_types=[
            pltpu.VMEM((_BPW, 50), jnp.int32),
            pltpu.VMEM((50, _DIMP), jnp.float32),
            pltpu.VMEM((50, _DIMP), jnp.float32),
            pltpu.SemaphoreType.DMA,
            pltpu.SemaphoreType.DMA,
        ],
    )
    return f(inp, table, tail)


def kernel(inp, table):
    tail = jnp.pad(
        lax.slice(table, (0, 256), (table.shape[0], _DIM)),
        ((0, 0), (0, 128 - (_DIM - 256))),
    )
    y = _lookup(inp, table, tail)
    return y[:, :, :_DIM]
